# baseline (speedup 1.0000x reference)
import functools

import jax
import jax.numpy as jnp
from jax import lax
from jax.experimental import pallas as pl
from jax.experimental.pallas import tpu as pltpu
from jax.experimental.pallas import tpu_sc as plsc

N = 10000
E = 320000
D_IN = 512
D_H = 128
D_G = 128
C = 16

NPAD = 10240
NC, NS, L = 2, 16, 16
NW = NC * NS
B1 = 64
NB1 = E // B1
B2 = 128
NB2 = E // B2
BR = 320
NBR = E // BR
RPT = NPAD // NS


def _front_body(x_ref, w_ref, h_ref, hg_ref):
    y = jnp.dot(x_ref[...], w_ref[...], preferred_element_type=jnp.float32)
    h = jnp.maximum(y[:, :D_H], 0.0)
    h_ref[...] = h
    hg_ref[...] = jnp.concatenate([h, y[:, D_H:]], axis=1)


def _front(x, wcat):
    bn = 1000
    return pl.pallas_call(
        _front_body,
        grid=(N // bn,),
        in_specs=[
            pl.BlockSpec((bn, D_IN), lambda i: (i, 0)),
            pl.BlockSpec((D_IN, D_H + D_G), lambda i: (0, 0)),
        ],
        out_specs=[
            pl.BlockSpec((bn, D_H), lambda i: (i, 0)),
            pl.BlockSpec((bn, D_H + D_G), lambda i: (i, 0)),
        ],
        out_shape=[
            jax.ShapeDtypeStruct((N, D_H), jnp.float32),
            jax.ShapeDtypeStruct((N, D_H + D_G), jnp.float32),
        ],
    )(x, wcat)


def _mid_body(rsp_ref, y1p_ref, inv_ref, x1_ref):
    rs = rsp_ref[0, :, 0] + rsp_ref[1, :, 0]
    inv = 1.0 / (rs + 1e-16)
    inv_ref[0, :] = inv
    y1 = y1p_ref[0] + y1p_ref[1]
    x1_ref[...] = jnp.maximum(y1 * inv[:, None], 0.0)


def _mid(rsp, y1p):
    bn = 1024
    return pl.pallas_call(
        _mid_body,
        grid=(NPAD // bn,),
        in_specs=[
            pl.BlockSpec((2, bn, D_H), lambda i: (0, i, 0)),
            pl.BlockSpec((2, bn, D_G), lambda i: (0, i, 0)),
        ],
        out_specs=[
            pl.BlockSpec((1, bn), lambda i: (0, i)),
            pl.BlockSpec((bn, D_G), lambda i: (i, 0)),
        ],
        out_shape=[
            jax.ShapeDtypeStruct((1, NPAD), jnp.float32),
            jax.ShapeDtypeStruct((NPAD, D_G), jnp.float32),
        ],
    )(rsp, y1p)


def _tail_body(up_ref, w2_ref, out_ref):
    u = up_ref[0] + up_ref[1]
    out_ref[...] = jnp.dot(u, w2_ref[...], preferred_element_type=jnp.float32)


def _tail(up, w2):
    bn = 2000
    return pl.pallas_call(
        _tail_body,
        grid=(N // bn,),
        in_specs=[
            pl.BlockSpec((2, bn, D_G), lambda i: (0, i, 0)),
            pl.BlockSpec((D_G, C), lambda i: (0, 0)),
        ],
        out_specs=pl.BlockSpec((bn, C), lambda i: (i, 0)),
        out_shape=jax.ShapeDtypeStruct((N, C), jnp.float32),
    )(up, w2)


def _edge1_body(h_hbm, hg_hbm, src_hbm, dst_hbm, a_hbm,
                expv_hbm, y1p_hbm,
                a_v, sidx, didx, hs, hgd,
                wrow, expvb0, expvb1,
                acc_sh, sem_g, sem_i, sem_w0, sem_w1):
    cid = lax.axis_index("c")
    sid = lax.axis_index("s")
    wid = sid * NC + cid
    r0 = sid * RPT
    expvb = (expvb0, expvb1)
    sem_w = (sem_w0, sem_w1)

    pltpu.sync_copy(a_hbm, a_v)
    a_regs = [a_v[pl.ds(r * L, L)] for r in range(D_H // L)]

    def _zero_half(b):
        def _z1(i, c):
            for r in range(D_H // L):
                wrow[b * B1 + i, pl.ds(r * L, L)] = jnp.zeros((L,),
                                                              jnp.float32)
            return c

        lax.fori_loop(0, B1, _z1, 0)

    _zero_half(0)
    _zero_half(1)

    for k in range(RPT // (2 * B1)):
        pltpu.sync_copy(wrow, acc_sh.at[pl.ds(r0 + k * 2 * B1, 2 * B1), :])
    plsc.subcore_barrier()

    lanes = lax.iota(jnp.int32, L)

    nb = jnp.where(wid < NB1 % NW, NB1 // NW + 1, NB1 // NW)

    def _drain(b):
        pltpu.make_async_copy(expvb[b], expv_hbm.at[pl.ds(0, B1)],
                              sem_w[b]).wait()

    def _wait_idx(b):
        pltpu.make_async_copy(src_hbm.at[pl.ds(0, B1)],
                              sidx.at[pl.ds(b * B1, B1)], sem_i).wait()
        pltpu.make_async_copy(dst_hbm.at[pl.ds(0, B1)],
                              didx.at[pl.ds(b * B1, B1)], sem_i).wait()

    def _do(t, b, prefetch_other):
        cp1 = pltpu.async_copy(h_hbm.at[sidx.at[pl.ds(b * B1, B1)]],
                               hs, sem_g)
        cp2 = pltpu.async_copy(hg_hbm.at[didx.at[pl.ds(b * B1, B1)]],
                               hgd, sem_g)

        @pl.when(t >= 2)
        def _():
            _drain(b)

        if prefetch_other:
            off1 = (wid + (t + 1) * NW) * B1
            pltpu.async_copy(src_hbm.at[pl.ds(off1, B1)],
                             sidx.at[pl.ds((1 - b) * B1, B1)], sem_i)
            pltpu.async_copy(dst_hbm.at[pl.ds(off1, B1)],
                             didx.at[pl.ds((1 - b) * B1, B1)], sem_i)

        cp1.wait()
        cp2.wait()

        def _group(gi, cc):
            sv = jnp.zeros((L,), jnp.float32)
            for j in range(L):
                e = gi * L + j
                acc = jnp.zeros((L,), jnp.float32)
                for r in range(D_H // L):
                    vs = hs[e, pl.ds(r * L, L)]
                    vd = hgd[e, pl.ds(r * L, L)]
                    acc = acc + jnp.abs(vs - vd) * a_regs[r]
                sv = jnp.where(lanes == j, jnp.sum(acc), sv)
            ex = jnp.exp(jnp.maximum(sv, 0.0))
            expvb[b][pl.ds(gi * L, L)] = ex
            for j in range(L):
                e = gi * L + j
                w = jnp.broadcast_to(ex[j], (L,))
                for r in range(D_H // L):
                    wrow[b * B1 + e, pl.ds(r * L, L)] = \
                        hgd[e, pl.ds(D_H + r * L, L)] * w
            return cc

        lax.fori_loop(0, B1 // L, _group, 0)

        off = (wid + t * NW) * B1
        pltpu.async_copy(expvb[b], expv_hbm.at[pl.ds(off, B1)], sem_w[b])

    pltpu.sync_copy(src_hbm.at[pl.ds(wid * B1, B1)],
                    sidx.at[pl.ds(0, B1)])
    pltpu.sync_copy(dst_hbm.at[pl.ds(wid * B1, B1)],
                    didx.at[pl.ds(0, B1)])

    def _pair(p, c):
        t = 2 * p

        @pl.when(t >= 1)
        def _():
            _wait_idx(0)

        _do(t, 0, True)
        _wait_idx(1)
        _do(t + 1, 1, False)
        pltpu.sync_copy(wrow, acc_sh.at[sidx], add=True)
        off0 = (wid + jnp.minimum(t + 2, nb - 1) * NW) * B1
        pltpu.async_copy(src_hbm.at[pl.ds(off0, B1)],
                         sidx.at[pl.ds(0, B1)], sem_i)
        pltpu.async_copy(dst_hbm.at[pl.ds(off0, B1)],
                         didx.at[pl.ds(0, B1)], sem_i)
        return c

    lax.fori_loop(0, nb // 2, _pair, 0)

    _wait_idx(0)

    @pl.when(nb % 2 == 1)
    def _():
        _do(nb - 1, 0, False)
        _zero_half(1)
        pltpu.sync_copy(wrow, acc_sh.at[sidx], add=True)

    _drain(0)
    _drain(1)

    plsc.subcore_barrier()
    pltpu.sync_copy(acc_sh.at[pl.ds(r0, RPT), :],
                    y1p_hbm.at[cid, pl.ds(r0, RPT), :])


def _edge1(h, hg, src, dst, a):
    mesh = plsc.VectorSubcoreMesh(core_axis_name="c", subcore_axis_name="s")
    f = pl.kernel(
        _edge1_body,
        out_type=[
            jax.ShapeDtypeStruct((E,), jnp.float32),
            jax.ShapeDtypeStruct((NC, NPAD, D_G), jnp.float32),
        ],
        mesh=mesh,
        scratch_types=[
            pltpu.VMEM((D_H,), jnp.float32),
            pltpu.VMEM((2 * B1,), jnp.int32),
            pltpu.VMEM((2 * B1,), jnp.int32),
            pltpu.VMEM((B1, D_H), jnp.float32),
            pltpu.VMEM((B1, 2 * D_H), jnp.float32),
            pltpu.VMEM((2 * B1, D_H), jnp.float32),
            pltpu.VMEM((B1,), jnp.float32),
            pltpu.VMEM((B1,), jnp.float32),
            pltpu.VMEM_SHARED((NPAD, D_G), jnp.float32),
            pltpu.SemaphoreType.DMA,
            pltpu.SemaphoreType.DMA,
            pltpu.SemaphoreType.DMA,
            pltpu.SemaphoreType.DMA,
        ],
        compiler_params=pltpu.CompilerParams(needs_layout_passes=False),
    )
    return f(h, hg, src, dst, a)


def _rs_body(src_hbm, expv_hbm, rsp_hbm,
             sidx, expvb, wrs, rs_sh):
    cid = lax.axis_index("c")
    sid = lax.axis_index("s")
    wid = sid * NC + cid
    r0 = sid * RPT

    def _z1(i, c):
        for r in range(D_H // L):
            wrs[i, pl.ds(r * L, L)] = jnp.zeros((L,), jnp.float32)
        return c

    lax.fori_loop(0, BR, _z1, 0)

    for k in range(RPT // BR):
        pltpu.sync_copy(wrs, rs_sh.at[pl.ds(r0 + k * BR, BR), :])
    plsc.subcore_barrier()

    nb = jnp.where(wid < NBR % NW, NBR // NW + 1, NBR // NW)

    def _batch(t, c):
        off = (wid + t * NW) * BR
        pltpu.sync_copy(src_hbm.at[pl.ds(off, BR)], sidx)
        pltpu.sync_copy(expv_hbm.at[pl.ds(off, BR)], expvb)

        def _group(gi, cc):
            ex = expvb[pl.ds(gi * L, L)]
            for j in range(L):
                e = gi * L + j
                wrs[e, pl.ds(0, L)] = jnp.broadcast_to(ex[j], (L,))
            return cc

        lax.fori_loop(0, BR // L, _group, 0)
        pltpu.sync_copy(wrs, rs_sh.at[sidx], add=True)
        return c

    lax.fori_loop(0, nb, _batch, 0)
    plsc.subcore_barrier()
    pltpu.sync_copy(rs_sh.at[pl.ds(r0, RPT), :],
                    rsp_hbm.at[cid, pl.ds(r0, RPT), :])


def _rs(src, expv):
    mesh = plsc.VectorSubcoreMesh(core_axis_name="c", subcore_axis_name="s")
    f = pl.kernel(
        _rs_body,
        out_type=[
            jax.ShapeDtypeStruct((NC, NPAD, D_H), jnp.float32),
        ],
        mesh=mesh,
        scratch_types=[
            pltpu.VMEM((BR,), jnp.int32),
            pltpu.VMEM((BR,), jnp.float32),
            pltpu.VMEM((BR, D_H), jnp.float32),
            pltpu.VMEM_SHARED((NPAD, D_H), jnp.float32),
        ],
        compiler_params=pltpu.CompilerParams(needs_layout_passes=False),
    )
    return f(src, expv)


def _edge2_body(expv_hbm, src_hbm, dst_hbm, inv_hbm, x1_hbm,
                s_hbm, up_hbm,
                inv_v, sidx0, didx0, expvb0, sb0, sidx1, didx1, expvb1, sb1,
                x1d, wz,
                u_sh, sem_g, sem_i, sem_s0, sem_s1):
    cid = lax.axis_index("c")
    sid = lax.axis_index("s")
    wid = sid * NC + cid
    r0 = sid * RPT
    sidx = (sidx0, sidx1)
    didx = (didx0, didx1)
    expvb = (expvb0, expvb1)
    sb = (sb0, sb1)
    sem_s = (sem_s0, sem_s1)

    pltpu.sync_copy(inv_hbm, inv_v)

    def _z1(i, c):
        for r in range(D_G // L):
            wz[i, pl.ds(r * L, L)] = jnp.zeros((L,), jnp.float32)
        return c

    lax.fori_loop(0, B2, _z1, 0)
    for k in range(RPT // B2):
        pltpu.sync_copy(wz, u_sh.at[pl.ds(r0 + k * B2, B2), :])
    plsc.subcore_barrier()

    nb = jnp.where(wid < NB2 % NW, NB2 // NW + 1, NB2 // NW)

    def _drain(b):
        pltpu.make_async_copy(sb[b], s_hbm.at[pl.ds(0, B2)],
                              sem_s[b]).wait()

    def _do(t, b):
        cp = pltpu.async_copy(x1_hbm.at[didx[b]], x1d, sem_g)

        @pl.when(t >= 1)
        def _():
            _drain(1 - b)

        off1 = jnp.minimum(t + 1, nb - 1)
        off1 = (wid + off1 * NW) * B2
        pltpu.async_copy(src_hbm.at[pl.ds(off1, B2)], sidx[1 - b], sem_i)
        pltpu.async_copy(dst_hbm.at[pl.ds(off1, B2)], didx[1 - b], sem_i)
        pltpu.async_copy(expv_hbm.at[pl.ds(off1, B2)], expvb[1 - b], sem_i)

        cp.wait()

        def _group(gi, cc):
            iv = plsc.load_gather(inv_v, [sidx[b][pl.ds(gi * L, L)]])
            s16 = expvb[b][pl.ds(gi * L, L)] * iv
            sb[b][pl.ds(gi * L, L)] = s16
            for j in range(L):
                e = gi * L + j
                w = jnp.broadcast_to(s16[j], (L,))
                for r in range(D_G // L):
                    wz[e, pl.ds(r * L, L)] = x1d[e, pl.ds(r * L, L)] * w
            return cc

        lax.fori_loop(0, B2 // L, _group, 0)

        pltpu.make_async_copy(src_hbm.at[pl.ds(0, B2)], sidx[1 - b],
                              sem_i).wait()
        pltpu.make_async_copy(dst_hbm.at[pl.ds(0, B2)], didx[1 - b],
                              sem_i).wait()
        pltpu.make_async_copy(expv_hbm.at[pl.ds(0, B2)], expvb[1 - b],
                              sem_i).wait()

        off = (wid + t * NW) * B2
        pltpu.async_copy(sb[b], s_hbm.at[pl.ds(off, B2)], sem_s[b])
        pltpu.sync_copy(wz, u_sh.at[sidx[b]], add=True)

    pltpu.sync_copy(src_hbm.at[pl.ds(wid * B2, B2)], sidx0)
    pltpu.sync_copy(dst_hbm.at[pl.ds(wid * B2, B2)], didx0)
    pltpu.sync_copy(expv_hbm.at[pl.ds(wid * B2, B2)], expvb0)

    def _pair(p, c):
        _do(2 * p, 0)
        _do(2 * p + 1, 1)
        return c

    lax.fori_loop(0, nb // 2, _pair, 0)

    @pl.when(nb % 2 == 1)
    def _():
        _do(nb - 1, 0)

    @pl.when(nb % 2 == 1)
    def _():
        _drain(0)

    @pl.when(nb % 2 == 0)
    def _():
        _drain(1)

    plsc.subcore_barrier()
    pltpu.sync_copy(u_sh.at[pl.ds(r0, RPT), :],
                    up_hbm.at[cid, pl.ds(r0, RPT), :])


def _edge2(expv, src, dst, inv, x1):
    mesh = plsc.VectorSubcoreMesh(core_axis_name="c", subcore_axis_name="s")
    f = pl.kernel(
        _edge2_body,
        out_type=[
            jax.ShapeDtypeStruct((E,), jnp.float32),
            jax.ShapeDtypeStruct((NC, NPAD, D_G), jnp.float32),
        ],
        mesh=mesh,
        scratch_types=[
            pltpu.VMEM((NPAD,), jnp.float32),
            pltpu.VMEM((B2,), jnp.int32),
            pltpu.VMEM((B2,), jnp.int32),
            pltpu.VMEM((B2,), jnp.float32),
            pltpu.VMEM((B2,), jnp.float32),
            pltpu.VMEM((B2,), jnp.int32),
            pltpu.VMEM((B2,), jnp.int32),
            pltpu.VMEM((B2,), jnp.float32),
            pltpu.VMEM((B2,), jnp.float32),
            pltpu.VMEM((B2, D_G), jnp.float32),
            pltpu.VMEM((B2, D_G), jnp.float32),
            pltpu.VMEM_SHARED((NPAD, D_G), jnp.float32),
            pltpu.SemaphoreType.DMA,
            pltpu.SemaphoreType.DMA,
            pltpu.SemaphoreType.DMA,
            pltpu.SemaphoreType.DMA,
        ],
        compiler_params=pltpu.CompilerParams(needs_layout_passes=False),
    )
    return f(expv, src, dst, inv, x1)


def kernel(x, edge_index, W_sgl, a, W1, W2):
    src = edge_index[0]
    dst = edge_index[1]
    wcat = jnp.concatenate([W_sgl, W1], axis=1)
    h, hg = _front(x, wcat)
    expv, y1p = _edge1(h, hg, src, dst, a.reshape(D_H))
    (rsp,) = _rs(src, expv)
    inv2d, x1 = _mid(rsp, y1p)
    inv = inv2d.reshape(NPAD)
    s_vals, up = _edge2(expv, src, dst, inv, x1)
    logits = _tail(up, W2)
    return logits, h, s_vals


# speedup vs baseline: 1.0243x; 1.0243x over previous
import functools

import jax
import jax.numpy as jnp
from jax import lax
from jax.experimental import pallas as pl
from jax.experimental.pallas import tpu as pltpu
from jax.experimental.pallas import tpu_sc as plsc

N = 10000
E = 320000
D_IN = 512
D_H = 128
D_G = 128
C = 16

NPAD = 10240
NC, NS, L = 2, 16, 16
NW = NC * NS
B1 = 64
NB1 = E // B1
B2 = 128
NB2 = E // B2
BR = 320
NBR = E // BR
RPT = NPAD // NS


def _front_body(x_ref, w_ref, h_ref, hg_ref):
    y = jnp.dot(x_ref[...], w_ref[...], preferred_element_type=jnp.float32)
    h = jnp.maximum(y[:, :D_H], 0.0)
    h_ref[...] = h
    hg_ref[...] = jnp.concatenate([h, y[:, D_H:]], axis=1)


def _front(x, wcat):
    bn = 1000
    return pl.pallas_call(
        _front_body,
        grid=(N // bn,),
        in_specs=[
            pl.BlockSpec((bn, D_IN), lambda i: (i, 0)),
            pl.BlockSpec((D_IN, D_H + D_G), lambda i: (0, 0)),
        ],
        out_specs=[
            pl.BlockSpec((bn, D_H), lambda i: (i, 0)),
            pl.BlockSpec((bn, D_H + D_G), lambda i: (i, 0)),
        ],
        out_shape=[
            jax.ShapeDtypeStruct((N, D_H), jnp.float32),
            jax.ShapeDtypeStruct((N, D_H + D_G), jnp.float32),
        ],
    )(x, wcat)


def _mid_body(rsp_ref, y1p_ref, inv_ref, x1_ref):
    rs = rsp_ref[0, :, 0] + rsp_ref[1, :, 0]
    inv = 1.0 / (rs + 1e-16)
    inv_ref[0, :] = inv
    y1 = y1p_ref[0] + y1p_ref[1]
    x1_ref[...] = jnp.maximum(y1 * inv[:, None], 0.0)


def _mid(rsp, y1p):
    bn = 1024
    return pl.pallas_call(
        _mid_body,
        grid=(NPAD // bn,),
        in_specs=[
            pl.BlockSpec((2, bn, D_H), lambda i: (0, i, 0)),
            pl.BlockSpec((2, bn, D_G), lambda i: (0, i, 0)),
        ],
        out_specs=[
            pl.BlockSpec((1, bn), lambda i: (0, i)),
            pl.BlockSpec((bn, D_G), lambda i: (i, 0)),
        ],
        out_shape=[
            jax.ShapeDtypeStruct((1, NPAD), jnp.float32),
            jax.ShapeDtypeStruct((NPAD, D_G), jnp.float32),
        ],
    )(rsp, y1p)


def _tail_body(up_ref, w2_ref, out_ref):
    u = up_ref[0] + up_ref[1]
    out_ref[...] = jnp.dot(u, w2_ref[...], preferred_element_type=jnp.float32)


def _tail(up, w2):
    bn = 2000
    return pl.pallas_call(
        _tail_body,
        grid=(N // bn,),
        in_specs=[
            pl.BlockSpec((2, bn, D_G), lambda i: (0, i, 0)),
            pl.BlockSpec((D_G, C), lambda i: (0, 0)),
        ],
        out_specs=pl.BlockSpec((bn, C), lambda i: (i, 0)),
        out_shape=jax.ShapeDtypeStruct((N, C), jnp.float32),
    )(up, w2)


def _edge1_body(h_hbm, hg_hbm, src_hbm, dst_hbm, a_hbm,
                expv_hbm, y1p_hbm,
                a_v, sidx0, didx0, sidx1, didx1, hs, hgd,
                wrow0, expvb0, wrow1, expvb1,
                acc_sh, sem_g, sem_i, sem_w0, sem_w1):
    cid = lax.axis_index("c")
    sid = lax.axis_index("s")
    wid = sid * NC + cid
    r0 = sid * RPT
    sidx = (sidx0, sidx1)
    didx = (didx0, didx1)
    wrow = (wrow0, wrow1)
    expvb = (expvb0, expvb1)
    sem_w = (sem_w0, sem_w1)

    pltpu.sync_copy(a_hbm, a_v)
    a_regs = [a_v[pl.ds(r * L, L)] for r in range(D_H // L)]

    def _z1(i, c):
        for r in range(D_H // L):
            wrow0[i, pl.ds(r * L, L)] = jnp.zeros((L,), jnp.float32)
        return c

    lax.fori_loop(0, B1, _z1, 0)

    for k in range(RPT // B1):
        pltpu.sync_copy(wrow0, acc_sh.at[pl.ds(r0 + k * B1, B1), :])
    plsc.subcore_barrier()

    lanes = lax.iota(jnp.int32, L)

    nb = jnp.where(wid < NB1 % NW, NB1 // NW + 1, NB1 // NW)

    def _drain(b):
        pltpu.make_async_copy(expvb[b], expv_hbm.at[pl.ds(0, B1)],
                              sem_w[b]).wait()

    def _do(t, b):
        cp1 = pltpu.async_copy(h_hbm.at[sidx[b]], hs, sem_g)
        cp2 = pltpu.async_copy(hg_hbm.at[didx[b]], hgd, sem_g)

        @pl.when(t >= 1)
        def _():
            _drain(1 - b)

        off1 = jnp.minimum(t + 1, nb - 1)
        off1 = (wid + off1 * NW) * B1
        pltpu.async_copy(src_hbm.at[pl.ds(off1, B1)], sidx[1 - b], sem_i)
        pltpu.async_copy(dst_hbm.at[pl.ds(off1, B1)], didx[1 - b], sem_i)

        cp1.wait()
        cp2.wait()

        def _group(gi, cc):
            sv = jnp.zeros((L,), jnp.float32)
            for j in range(L):
                e = gi * L + j
                acc = jnp.zeros((L,), jnp.float32)
                for r in range(D_H // L):
                    vs = hs[e, pl.ds(r * L, L)]
                    vd = hgd[e, pl.ds(r * L, L)]
                    acc = acc + jnp.abs(vs - vd) * a_regs[r]
                sv = jnp.where(lanes == j, jnp.sum(acc), sv)
            ex = jnp.exp(jnp.maximum(sv, 0.0))
            expvb[b][pl.ds(gi * L, L)] = ex
            for j in range(L):
                e = gi * L + j
                w = jnp.broadcast_to(ex[j], (L,))
                for r in range(D_H // L):
                    wrow[b][e, pl.ds(r * L, L)] = \
                        hgd[e, pl.ds(D_H + r * L, L)] * w
            return cc

        lax.fori_loop(0, B1 // L, _group, 0)

        pltpu.make_async_copy(src_hbm.at[pl.ds(0, B1)], sidx[1 - b],
                              sem_i).wait()
        pltpu.make_async_copy(dst_hbm.at[pl.ds(0, B1)], didx[1 - b],
                              sem_i).wait()

        off = (wid + t * NW) * B1
        pltpu.async_copy(expvb[b], expv_hbm.at[pl.ds(off, B1)], sem_w[b])
        pltpu.sync_copy(wrow[b], acc_sh.at[sidx[b]], add=True)

    pltpu.sync_copy(src_hbm.at[pl.ds(wid * B1, B1)], sidx0)
    pltpu.sync_copy(dst_hbm.at[pl.ds(wid * B1, B1)], didx0)

    def _pair(p, c):
        _do(2 * p, 0)
        _do(2 * p + 1, 1)
        return c

    lax.fori_loop(0, nb // 2, _pair, 0)

    @pl.when(nb % 2 == 1)
    def _():
        _do(nb - 1, 0)

    @pl.when(nb % 2 == 1)
    def _():
        _drain(0)

    @pl.when(nb % 2 == 0)
    def _():
        _drain(1)

    plsc.subcore_barrier()
    pltpu.sync_copy(acc_sh.at[pl.ds(r0, RPT), :],
                    y1p_hbm.at[cid, pl.ds(r0, RPT), :])


def _edge1(h, hg, src, dst, a):
    mesh = plsc.VectorSubcoreMesh(core_axis_name="c", subcore_axis_name="s")
    f = pl.kernel(
        _edge1_body,
        out_type=[
            jax.ShapeDtypeStruct((E,), jnp.float32),
            jax.ShapeDtypeStruct((NC, NPAD, D_G), jnp.float32),
        ],
        mesh=mesh,
        scratch_types=[
            pltpu.VMEM((D_H,), jnp.float32),
            pltpu.VMEM((B1,), jnp.int32),
            pltpu.VMEM((B1,), jnp.int32),
            pltpu.VMEM((B1,), jnp.int32),
            pltpu.VMEM((B1,), jnp.int32),
            pltpu.VMEM((B1, D_H), jnp.float32),
            pltpu.VMEM((B1, 2 * D_H), jnp.float32),
            pltpu.VMEM((B1, D_H), jnp.float32),
            pltpu.VMEM((B1,), jnp.float32),
            pltpu.VMEM((B1, D_H), jnp.float32),
            pltpu.VMEM((B1,), jnp.float32),
            pltpu.VMEM_SHARED((NPAD, D_G), jnp.float32),
            pltpu.SemaphoreType.DMA,
            pltpu.SemaphoreType.DMA,
            pltpu.SemaphoreType.DMA,
            pltpu.SemaphoreType.DMA,
        ],
        compiler_params=pltpu.CompilerParams(needs_layout_passes=False),
    )
    return f(h, hg, src, dst, a)


def _rs_body(src_hbm, expv_hbm, rsp_hbm,
             sidx, expvb, wrs, rs_sh):
    cid = lax.axis_index("c")
    sid = lax.axis_index("s")
    wid = sid * NC + cid
    r0 = sid * RPT

    def _z1(i, c):
        for r in range(D_H // L):
            wrs[i, pl.ds(r * L, L)] = jnp.zeros((L,), jnp.float32)
        return c

    lax.fori_loop(0, BR, _z1, 0)

    for k in range(RPT // BR):
        pltpu.sync_copy(wrs, rs_sh.at[pl.ds(r0 + k * BR, BR), :])
    plsc.subcore_barrier()

    nb = jnp.where(wid < NBR % NW, NBR // NW + 1, NBR // NW)

    def _batch(t, c):
        off = (wid + t * NW) * BR
        pltpu.sync_copy(src_hbm.at[pl.ds(off, BR)], sidx)
        pltpu.sync_copy(expv_hbm.at[pl.ds(off, BR)], expvb)

        def _group(gi, cc):
            ex = expvb[pl.ds(gi * L, L)]
            for j in range(L):
                e = gi * L + j
                wrs[e, pl.ds(0, L)] = jnp.broadcast_to(ex[j], (L,))
            return cc

        lax.fori_loop(0, BR // L, _group, 0)
        pltpu.sync_copy(wrs, rs_sh.at[sidx], add=True)
        return c

    lax.fori_loop(0, nb, _batch, 0)
    plsc.subcore_barrier()
    pltpu.sync_copy(rs_sh.at[pl.ds(r0, RPT), :],
                    rsp_hbm.at[cid, pl.ds(r0, RPT), :])


def _rs(src, expv):
    mesh = plsc.VectorSubcoreMesh(core_axis_name="c", subcore_axis_name="s")
    f = pl.kernel(
        _rs_body,
        out_type=[
            jax.ShapeDtypeStruct((NC, NPAD, D_H), jnp.float32),
        ],
        mesh=mesh,
        scratch_types=[
            pltpu.VMEM((BR,), jnp.int32),
            pltpu.VMEM((BR,), jnp.float32),
            pltpu.VMEM((BR, D_H), jnp.float32),
            pltpu.VMEM_SHARED((NPAD, D_H), jnp.float32),
        ],
        compiler_params=pltpu.CompilerParams(needs_layout_passes=False),
    )
    return f(src, expv)


def _edge2_body(expv_hbm, src_hbm, dst_hbm, inv_hbm, x1_hbm,
                s_hbm, up_hbm,
                inv_v, sidx0, didx0, expvb0, sb0, sidx1, didx1, expvb1, sb1,
                x1d, wz,
                u_sh, sem_g, sem_i, sem_s0, sem_s1):
    cid = lax.axis_index("c")
    sid = lax.axis_index("s")
    wid = sid * NC + cid
    r0 = sid * RPT
    sidx = (sidx0, sidx1)
    didx = (didx0, didx1)
    expvb = (expvb0, expvb1)
    sb = (sb0, sb1)
    sem_s = (sem_s0, sem_s1)

    pltpu.sync_copy(inv_hbm, inv_v)

    def _z1(i, c):
        for r in range(D_G // L):
            wz[i, pl.ds(r * L, L)] = jnp.zeros((L,), jnp.float32)
        return c

    lax.fori_loop(0, B2, _z1, 0)
    for k in range(RPT // B2):
        pltpu.sync_copy(wz, u_sh.at[pl.ds(r0 + k * B2, B2), :])
    plsc.subcore_barrier()

    nb = jnp.where(wid < NB2 % NW, NB2 // NW + 1, NB2 // NW)

    def _drain(b):
        pltpu.make_async_copy(sb[b], s_hbm.at[pl.ds(0, B2)],
                              sem_s[b]).wait()

    def _do(t, b):
        cp = pltpu.async_copy(x1_hbm.at[didx[b]], x1d, sem_g)

        @pl.when(t >= 1)
        def _():
            _drain(1 - b)

        off1 = jnp.minimum(t + 1, nb - 1)
        off1 = (wid + off1 * NW) * B2
        pltpu.async_copy(src_hbm.at[pl.ds(off1, B2)], sidx[1 - b], sem_i)
        pltpu.async_copy(dst_hbm.at[pl.ds(off1, B2)], didx[1 - b], sem_i)
        pltpu.async_copy(expv_hbm.at[pl.ds(off1, B2)], expvb[1 - b], sem_i)

        cp.wait()

        def _group(gi, cc):
            iv = plsc.load_gather(inv_v, [sidx[b][pl.ds(gi * L, L)]])
            s16 = expvb[b][pl.ds(gi * L, L)] * iv
            sb[b][pl.ds(gi * L, L)] = s16
            for j in range(L):
                e = gi * L + j
                w = jnp.broadcast_to(s16[j], (L,))
                for r in range(D_G // L):
                    wz[e, pl.ds(r * L, L)] = x1d[e, pl.ds(r * L, L)] * w
            return cc

        lax.fori_loop(0, B2 // L, _group, 0)

        pltpu.make_async_copy(src_hbm.at[pl.ds(0, B2)], sidx[1 - b],
                              sem_i).wait()
        pltpu.make_async_copy(dst_hbm.at[pl.ds(0, B2)], didx[1 - b],
                              sem_i).wait()
        pltpu.make_async_copy(expv_hbm.at[pl.ds(0, B2)], expvb[1 - b],
                              sem_i).wait()

        off = (wid + t * NW) * B2
        pltpu.async_copy(sb[b], s_hbm.at[pl.ds(off, B2)], sem_s[b])
        pltpu.sync_copy(wz, u_sh.at[sidx[b]], add=True)

    pltpu.sync_copy(src_hbm.at[pl.ds(wid * B2, B2)], sidx0)
    pltpu.sync_copy(dst_hbm.at[pl.ds(wid * B2, B2)], didx0)
    pltpu.sync_copy(expv_hbm.at[pl.ds(wid * B2, B2)], expvb0)

    def _pair(p, c):
        _do(2 * p, 0)
        _do(2 * p + 1, 1)
        return c

    lax.fori_loop(0, nb // 2, _pair, 0)

    @pl.when(nb % 2 == 1)
    def _():
        _do(nb - 1, 0)

    @pl.when(nb % 2 == 1)
    def _():
        _drain(0)

    @pl.when(nb % 2 == 0)
    def _():
        _drain(1)

    plsc.subcore_barrier()
    pltpu.sync_copy(u_sh.at[pl.ds(r0, RPT), :],
                    up_hbm.at[cid, pl.ds(r0, RPT), :])


def _edge2(expv, src, dst, inv, x1):
    mesh = plsc.VectorSubcoreMesh(core_axis_name="c", subcore_axis_name="s")
    f = pl.kernel(
        _edge2_body,
        out_type=[
            jax.ShapeDtypeStruct((E,), jnp.float32),
            jax.ShapeDtypeStruct((NC, NPAD, D_G), jnp.float32),
        ],
        mesh=mesh,
        scratch_types=[
            pltpu.VMEM((NPAD,), jnp.float32),
            pltpu.VMEM((B2,), jnp.int32),
            pltpu.VMEM((B2,), jnp.int32),
            pltpu.VMEM((B2,), jnp.float32),
            pltpu.VMEM((B2,), jnp.float32),
            pltpu.VMEM((B2,), jnp.int32),
            pltpu.VMEM((B2,), jnp.int32),
            pltpu.VMEM((B2,), jnp.float32),
            pltpu.VMEM((B2,), jnp.float32),
            pltpu.VMEM((B2, D_G), jnp.float32),
            pltpu.VMEM((B2, D_G), jnp.float32),
            pltpu.VMEM_SHARED((NPAD, D_G), jnp.float32),
            pltpu.SemaphoreType.DMA,
            pltpu.SemaphoreType.DMA,
            pltpu.SemaphoreType.DMA,
            pltpu.SemaphoreType.DMA,
        ],
        compiler_params=pltpu.CompilerParams(needs_layout_passes=False),
    )
    return f(expv, src, dst, inv, x1)


def kernel(x, edge_index, W_sgl, a, W1, W2):
    src = edge_index[0]
    dst = edge_index[1]
    wcat = jnp.concatenate([W_sgl, W1], axis=1)
    h, hg = _front(x, wcat)
    expv, y1p = _edge1(h, hg, src, dst, a.reshape(D_H))
    (rsp,) = _rs(src, expv)
    inv2d, x1 = _mid(rsp, y1p)
    inv = inv2d.reshape(NPAD)
    s_vals, up = _edge2(expv, src, dst, inv, x1)
    logits = _tail(up, W2)
    return logits, h, s_vals
